# baseline (speedup 1.0000x reference)
import functools
import numpy as np
import jax
import jax.numpy as jnp
from jax.experimental import pallas as pl
from jax.experimental.pallas import tpu as pltpu

_G = 8


def _lora_body(n_steps, xids_ref, wids_ref, rows_ref, *refs):
    x_refs = refs[0:_G]
    a_refs = refs[_G:2 * _G]
    b_refs = refs[2 * _G:3 * _G]
    out_ref = refs[3 * _G]
    i = pl.program_id(0)

    @pl.when(i == 0)
    def _init():
        out_ref[...] = jnp.zeros_like(out_ref)

    for k in range(_G):
        xr = x_refs[k][0]
        at = a_refs[k][0]
        b = b_refs[k][0]
        t = at * xr
        v = jnp.sum(t, axis=1, keepdims=True) * 2.0
        y = jax.lax.dot_general(v, b, (((0,), (0,)), ((), ())),
                                preferred_element_type=jnp.float32)
        row = rows_ref[k * n_steps + i]
        out_ref[pl.ds(row, 1), :] += y


def kernel(x, xids, wids, lora_A, lora_B):
    batch, _, d_model = x.shape
    lora_batch = xids.shape[0]
    r = lora_A.shape[2]

    r_mult = 4
    batch_large = (lora_batch - batch) // (r_mult - 1)
    n_summed = batch_large * r_mult
    n_steps = lora_batch // _G

    ar = np.arange(lora_batch)
    rows_static = jnp.asarray(
        np.where(ar < n_summed, ar // r_mult, ar - n_summed + batch_large)
        .astype(np.int32))

    order = jnp.argsort(wids)
    s_wids = wids[order]
    s_xids = xids[order]
    s_rows = rows_static[order]

    at = lora_A.transpose(0, 2, 1)

    def x_spec(k):
        return pl.BlockSpec(
            (1, 1, d_model),
            lambda i, xids, wids, rows, k=k: (xids[k * n_steps + i], 0, 0))

    def ab_spec(k):
        return pl.BlockSpec(
            (1, r, d_model),
            lambda i, xids, wids, rows, k=k: (wids[k * n_steps + i], 0, 0))

    grid_spec = pltpu.PrefetchScalarGridSpec(
        num_scalar_prefetch=3,
        grid=(n_steps,),
        in_specs=(
            [x_spec(k) for k in range(_G)]
            + [ab_spec(k) for k in range(_G)]
            + [ab_spec(k) for k in range(_G)]
        ),
        out_specs=pl.BlockSpec((batch, d_model),
                               lambda i, xids, wids, rows: (0, 0)),
    )
    out = pl.pallas_call(
        functools.partial(_lora_body, n_steps),
        grid_spec=grid_spec,
        out_shape=jax.ShapeDtypeStruct((batch, d_model), jnp.float32),
        compiler_params=pltpu.CompilerParams(
            dimension_semantics=("arbitrary",),
        ),
    )(s_xids, s_wids, s_rows,
      *([x] * _G), *([at] * _G), *([lora_B] * _G))
    return out.reshape(batch, 1, d_model)


# speedup vs baseline: 1.0098x; 1.0098x over previous
import functools
import numpy as np
import jax
import jax.numpy as jnp
from jax.experimental import pallas as pl
from jax.experimental.pallas import tpu as pltpu

_G = 8


def _lora_body(n_steps, xids_ref, wids_ref, rows_ref, *refs):
    x_refs = refs[0:_G]
    a_refs = refs[_G:2 * _G]
    b_refs = refs[2 * _G:3 * _G]
    out_ref = refs[3 * _G]
    i = pl.program_id(0)

    @pl.when(i == 0)
    def _init():
        out_ref[...] = jnp.zeros_like(out_ref)

    for k in range(_G):
        xr = x_refs[k][0]
        at = a_refs[k][0]
        b = b_refs[k][0]
        t = at * xr
        v = jnp.sum(t, axis=1, keepdims=True) * 2.0
        y = jax.lax.dot_general(v, b, (((0,), (0,)), ((), ())),
                                preferred_element_type=jnp.float32)
        row = rows_ref[k * n_steps + i]
        out_ref[pl.ds(row, 1), :] += y


def kernel(x, xids, wids, lora_A, lora_B):
    batch, _, d_model = x.shape
    lora_batch = xids.shape[0]
    r = lora_A.shape[2]

    r_mult = 4
    batch_large = (lora_batch - batch) // (r_mult - 1)
    n_summed = batch_large * r_mult
    n_steps = lora_batch // _G

    ar = np.arange(lora_batch)
    rows_static = jnp.asarray(
        np.where(ar < n_summed, ar // r_mult, ar - n_summed + batch_large)
        .astype(np.int32))

    s_wids = wids
    s_xids = xids
    s_rows = rows_static

    at = lora_A.transpose(0, 2, 1)

    def x_spec(k):
        return pl.BlockSpec(
            (1, 1, d_model),
            lambda i, xids, wids, rows, k=k: (xids[k * n_steps + i], 0, 0))

    def ab_spec(k):
        return pl.BlockSpec(
            (1, r, d_model),
            lambda i, xids, wids, rows, k=k: (wids[k * n_steps + i], 0, 0))

    grid_spec = pltpu.PrefetchScalarGridSpec(
        num_scalar_prefetch=3,
        grid=(n_steps,),
        in_specs=(
            [x_spec(k) for k in range(_G)]
            + [ab_spec(k) for k in range(_G)]
            + [ab_spec(k) for k in range(_G)]
        ),
        out_specs=pl.BlockSpec((batch, d_model),
                               lambda i, xids, wids, rows: (0, 0)),
    )
    out = pl.pallas_call(
        functools.partial(_lora_body, n_steps),
        grid_spec=grid_spec,
        out_shape=jax.ShapeDtypeStruct((batch, d_model), jnp.float32),
        compiler_params=pltpu.CompilerParams(
            dimension_semantics=("arbitrary",),
        ),
    )(s_xids, s_wids, s_rows,
      *([x] * _G), *([at] * _G), *([lora_B] * _G))
    return out.reshape(batch, 1, d_model)


# speedup vs baseline: 1.1974x; 1.1857x over previous
import functools
import numpy as np
import jax
import jax.numpy as jnp
from jax.experimental import pallas as pl
from jax.experimental.pallas import tpu as pltpu

_G = 16


def _lora_body(n_group_steps, batch_large, xids_ref, wids_ref, *refs):
    x_refs = refs[0:_G]
    a_refs = refs[_G:2 * _G]
    b_refs = refs[2 * _G:3 * _G]
    out_ref = refs[3 * _G]
    i = pl.program_id(0)

    ys = []
    for k in range(_G):
        xr = x_refs[k][0]
        at = a_refs[k][0]
        b = b_refs[k][0]
        t = at * xr
        v = jnp.sum(t, axis=1, keepdims=True) * 2.0
        y = jax.lax.dot_general(v, b, (((0,), (0,)), ((), ())),
                                preferred_element_type=jnp.float32)
        ys.append(y)

    @pl.when(i < n_group_steps)
    def _groups():
        row = 4 * i
        for g in range(_G // 4):
            out_ref[pl.ds(row + g, 1), :] = (
                (ys[4 * g] + ys[4 * g + 1]) + (ys[4 * g + 2] + ys[4 * g + 3]))

    @pl.when(i >= n_group_steps)
    def _passthrough():
        base = batch_large + (i - n_group_steps) * _G
        for k in range(_G):
            out_ref[pl.ds(base + k, 1), :] = ys[k]


def kernel(x, xids, wids, lora_A, lora_B):
    batch, _, d_model = x.shape
    lora_batch = xids.shape[0]
    r = lora_A.shape[2]

    r_mult = 4
    batch_large = (lora_batch - batch) // (r_mult - 1)
    n_summed = batch_large * r_mult
    n_steps = lora_batch // _G
    n_group_steps = n_summed // _G

    at = lora_A.transpose(0, 2, 1)

    def x_spec(k):
        return pl.BlockSpec(
            (1, 1, d_model),
            lambda i, xids, wids, k=k: (xids[_G * i + k], 0, 0))

    def ab_spec(k):
        return pl.BlockSpec(
            (1, r, d_model),
            lambda i, xids, wids, k=k: (wids[_G * i + k], 0, 0))

    grid_spec = pltpu.PrefetchScalarGridSpec(
        num_scalar_prefetch=2,
        grid=(n_steps,),
        in_specs=(
            [x_spec(k) for k in range(_G)]
            + [ab_spec(k) for k in range(_G)]
            + [ab_spec(k) for k in range(_G)]
        ),
        out_specs=pl.BlockSpec((batch, d_model),
                               lambda i, xids, wids: (0, 0)),
    )
    out = pl.pallas_call(
        functools.partial(_lora_body, n_group_steps, batch_large),
        grid_spec=grid_spec,
        out_shape=jax.ShapeDtypeStruct((batch, d_model), jnp.float32),
        compiler_params=pltpu.CompilerParams(
            dimension_semantics=("arbitrary",),
        ),
    )(xids, wids,
      *([x] * _G), *([at] * _G), *([lora_B] * _G))
    return out.reshape(batch, 1, d_model)


# speedup vs baseline: 1.2003x; 1.0024x over previous
import functools
import numpy as np
import jax
import jax.numpy as jnp
from jax.experimental import pallas as pl
from jax.experimental.pallas import tpu as pltpu

_G = 32


def _lora_body(n_group_steps, batch_large, xids_ref, wids_ref, *refs):
    x_refs = refs[0:_G]
    a_refs = refs[_G:2 * _G]
    b_refs = refs[2 * _G:3 * _G]
    out_ref = refs[3 * _G]
    i = pl.program_id(0)

    ys = []
    for k in range(_G):
        xr = x_refs[k][0]
        at = a_refs[k][0]
        b = b_refs[k][0]
        t = at * xr
        v = jnp.sum(t, axis=1, keepdims=True) * 2.0
        y = jax.lax.dot_general(v, b, (((0,), (0,)), ((), ())),
                                preferred_element_type=jnp.float32)
        ys.append(y)

    @pl.when(i < n_group_steps)
    def _groups():
        row = (_G // 4) * i
        for g in range(_G // 4):
            out_ref[pl.ds(row + g, 1), :] = (
                (ys[4 * g] + ys[4 * g + 1]) + (ys[4 * g + 2] + ys[4 * g + 3]))

    @pl.when(i >= n_group_steps)
    def _passthrough():
        base = batch_large + (i - n_group_steps) * _G
        for k in range(_G):
            out_ref[pl.ds(base + k, 1), :] = ys[k]


def kernel(x, xids, wids, lora_A, lora_B):
    batch, _, d_model = x.shape
    lora_batch = xids.shape[0]
    r = lora_A.shape[2]

    r_mult = 4
    batch_large = (lora_batch - batch) // (r_mult - 1)
    n_summed = batch_large * r_mult
    n_steps = lora_batch // _G
    n_group_steps = n_summed // _G

    at = lora_A.transpose(0, 2, 1)

    def x_spec(k):
        return pl.BlockSpec(
            (1, 1, d_model),
            lambda i, xids, wids, k=k: (xids[_G * i + k], 0, 0))

    def ab_spec(k):
        return pl.BlockSpec(
            (1, r, d_model),
            lambda i, xids, wids, k=k: (wids[_G * i + k], 0, 0))

    grid_spec = pltpu.PrefetchScalarGridSpec(
        num_scalar_prefetch=2,
        grid=(n_steps,),
        in_specs=(
            [x_spec(k) for k in range(_G)]
            + [ab_spec(k) for k in range(_G)]
            + [ab_spec(k) for k in range(_G)]
        ),
        out_specs=pl.BlockSpec((batch, d_model),
                               lambda i, xids, wids: (0, 0)),
    )
    out = pl.pallas_call(
        functools.partial(_lora_body, n_group_steps, batch_large),
        grid_spec=grid_spec,
        out_shape=jax.ShapeDtypeStruct((batch, d_model), jnp.float32),
        compiler_params=pltpu.CompilerParams(
            dimension_semantics=("arbitrary",),
        ),
    )(xids, wids,
      *([x] * _G), *([at] * _G), *([lora_B] * _G))
    return out.reshape(batch, 1, d_model)
